# baseline (speedup 1.0000x reference)
import functools
import math

import numpy as np
import jax
import jax.numpy as jnp
from jax import lax
from jax.experimental import pallas as pl

_NS = 16
_EMB = 64
_TINY = np.float32(np.finfo(np.float32).tiny)
_KS0 = np.uint32(0)
_KS1 = np.uint32(42)
_KS2 = np.uint32(0x1BD11BDA ^ 42)
_ROTS = ((13, 15, 26, 6), (17, 29, 16, 24))


def _threefry_bits(cnt):
    ks = (_KS0, _KS1, _KS2)
    x0 = jnp.zeros_like(cnt)
    x1 = cnt + _KS1
    for i in range(5):
        for r in _ROTS[i % 2]:
            x0 = x0 + x1
            x1 = (x1 << np.uint32(r)) | (x1 >> np.uint32(32 - r))
            x1 = x0 ^ x1
        x0 = x0 + ks[(i + 1) % 3]
        x1 = x1 + ks[(i + 2) % 3] + np.uint32(i + 1)
    return x0 ^ x1


def _sample_kernel(hm_ref, jt_ref, fr_ref, out_ref, *, j, hw, w, bj):
    b = pl.program_id(0)
    hm = hm_ref[0]
    probs = jnp.where(hm < 0.0, 0.0, hm)
    posm = probs > 0.0
    logits = jnp.where(posm, jnp.log(jnp.where(posm, probs, 1.0)),
                       -jnp.inf)
    col = lax.broadcasted_iota(jnp.int32, (j, hw), 1)
    row = lax.broadcasted_iota(jnp.int32, (j, hw), 0)
    base = b * (j * hw) + row * hw + col
    stride_s = bj * hw

    x_cols, y_cols, p_cols = [], [], []
    for s in range(_NS):
        cnt = (base + s * stride_s).astype(jnp.uint32)
        bits = _threefry_bits(cnt)
        fb = (bits >> np.uint32(9)) | np.uint32(0x3F800000)
        u = lax.bitcast_convert_type(fb, jnp.float32) - 1.0
        u = jnp.maximum(u, _TINY)
        g = -jnp.log(-jnp.log(u))
        score = g + logits
        m = jnp.max(score, axis=1, keepdims=True)
        idx = jnp.min(jnp.where(score == m, col, hw), axis=1,
                      keepdims=True)
        p = jnp.sum(jnp.where(col == idx, probs, 0.0), axis=1,
                    keepdims=True)
        q = idx // w
        x_cols.append((idx - q * w).astype(jnp.float32))
        y_cols.append(q.astype(jnp.float32))
        p_cols.append(p)

    X = jnp.concatenate(x_cols, axis=1)
    Y = jnp.concatenate(y_cols, axis=1)
    P = jnp.concatenate(p_cols, axis=1)

    n_tot = j * _NS
    xc = X - jnp.sum(X) / n_tot
    yc = Y - jnp.sum(Y) / n_tot
    m2 = (jnp.sum(xc) + jnp.sum(yc)) / (2 * n_tot)
    var = (jnp.sum((xc - m2) ** 2) + jnp.sum((yc - m2) ** 2)) / (2 * n_tot - 1)
    std = jnp.sqrt(var)
    xn = xc / std
    yn = yc / std

    d3 = lax.broadcasted_iota(jnp.int32, (j, _NS, 2 * _EMB), 2)
    coord = jnp.where(d3 < _EMB, xn[:, :, None], yn[:, :, None])
    freqs = fr_ref[...][None]
    arg = coord * freqs
    emb = jnp.where((d3 % _EMB) < (_EMB // 2), jnp.sin(arg), jnp.cos(arg))
    tok = emb * P[:, :, None] + jt_ref[...][:, None, :]
    out_ref[0] = tok


def kernel(heatmap, joint_table):
    b, j, h, w = heatmap.shape
    hw = h * w
    hm = heatmap.reshape(b, j, hw)

    half = _EMB // 2
    scale = math.log(10000.0) / (half - 1)
    fr = jnp.exp(jnp.arange(half, dtype=jnp.float32) * -scale)
    fr128 = jnp.concatenate([fr, fr, fr, fr]).reshape(1, 4 * half)

    out = pl.pallas_call(
        functools.partial(_sample_kernel, j=j, hw=hw, w=w, bj=b * j),
        grid=(b,),
        in_specs=[
            pl.BlockSpec((1, j, hw), lambda i: (i, 0, 0)),
            pl.BlockSpec((j, 2 * _EMB), lambda i: (0, 0)),
            pl.BlockSpec((1, 4 * half), lambda i: (0, 0)),
        ],
        out_specs=pl.BlockSpec((1, j, _NS, 2 * _EMB), lambda i: (i, 0, 0, 0)),
        out_shape=jax.ShapeDtypeStruct((b, j, _NS, 2 * _EMB), jnp.float32),
    )(hm, joint_table, fr128)
    return jnp.transpose(out, (0, 2, 1, 3))


# speedup vs baseline: 1.0000x; 1.0000x over previous
import functools
import math

import numpy as np
import jax
import jax.numpy as jnp
from jax import lax
from jax.experimental import pallas as pl
from jax.experimental.pallas import tpu as pltpu

_NS = 16
_EMB = 64
_TINY = np.float32(np.finfo(np.float32).tiny)
_KS0 = np.uint32(0)
_KS1 = np.uint32(42)
_KS2 = np.uint32(0x1BD11BDA ^ 42)
_ROTS = ((13, 15, 26, 6), (17, 29, 16, 24))


def _threefry_bits(cnt):
    ks = (_KS0, _KS1, _KS2)
    x0 = jnp.zeros_like(cnt)
    x1 = cnt + _KS1
    for i in range(5):
        for r in _ROTS[i % 2]:
            x0 = x0 + x1
            x1 = (x1 << np.uint32(r)) | (x1 >> np.uint32(32 - r))
            x1 = x0 ^ x1
        x0 = x0 + ks[(i + 1) % 3]
        x1 = x1 + ks[(i + 2) % 3] + np.uint32(i + 1)
    return x0 ^ x1


def _sample_kernel(hm_ref, jt_ref, fr_ref, out_ref, *, j, hw, w, bj):
    b = pl.program_id(0)
    hm = hm_ref[0]
    probs = jnp.where(hm < 0.0, 0.0, hm)
    posm = probs > 0.0
    logits = jnp.where(posm, jnp.log(jnp.where(posm, probs, 1.0)),
                       -jnp.inf)
    col = lax.broadcasted_iota(jnp.int32, (j, hw), 1)
    row = lax.broadcasted_iota(jnp.int32, (j, hw), 0)
    base = b * (j * hw) + row * hw + col
    stride_s = bj * hw

    x_cols, y_cols, p_cols = [], [], []
    for s in range(_NS):
        cnt = (base + s * stride_s).astype(jnp.uint32)
        bits = _threefry_bits(cnt)
        fb = (bits >> np.uint32(9)) | np.uint32(0x3F800000)
        u = lax.bitcast_convert_type(fb, jnp.float32) - 1.0
        u = jnp.maximum(u, _TINY)
        g = -jnp.log(-jnp.log(u))
        score = g + logits
        m = jnp.max(score, axis=1, keepdims=True)
        idx = jnp.min(jnp.where(score == m, col, hw), axis=1,
                      keepdims=True)
        p = jnp.sum(jnp.where(col == idx, probs, 0.0), axis=1,
                    keepdims=True)
        q = idx // w
        x_cols.append((idx - q * w).astype(jnp.float32))
        y_cols.append(q.astype(jnp.float32))
        p_cols.append(p)

    X = jnp.concatenate(x_cols, axis=1)
    Y = jnp.concatenate(y_cols, axis=1)
    P = jnp.concatenate(p_cols, axis=1)

    n_tot = j * _NS
    xc = X - jnp.sum(X) / n_tot
    yc = Y - jnp.sum(Y) / n_tot
    m2 = (jnp.sum(xc) + jnp.sum(yc)) / (2 * n_tot)
    var = (jnp.sum((xc - m2) ** 2) + jnp.sum((yc - m2) ** 2)) / (2 * n_tot - 1)
    std = jnp.sqrt(var)
    xn = xc / std
    yn = yc / std

    d3 = lax.broadcasted_iota(jnp.int32, (j, _NS, 2 * _EMB), 2)
    coord = jnp.where(d3 < _EMB, xn[:, :, None], yn[:, :, None])
    freqs = fr_ref[...][None]
    arg = coord * freqs
    emb = jnp.where((d3 % _EMB) < (_EMB // 2), jnp.sin(arg), jnp.cos(arg))
    tok = emb * P[:, :, None] + jt_ref[...][:, None, :]
    out_ref[0] = tok


def kernel(heatmap, joint_table):
    b, j, h, w = heatmap.shape
    hw = h * w
    hm = heatmap.reshape(b, j, hw)

    half = _EMB // 2
    scale = math.log(10000.0) / (half - 1)
    fr = jnp.exp(jnp.arange(half, dtype=jnp.float32) * -scale)
    fr128 = jnp.concatenate([fr, fr, fr, fr]).reshape(1, 4 * half)

    out = pl.pallas_call(
        functools.partial(_sample_kernel, j=j, hw=hw, w=w, bj=b * j),
        grid=(b,),
        in_specs=[
            pl.BlockSpec((1, j, hw), lambda i: (i, 0, 0)),
            pl.BlockSpec((j, 2 * _EMB), lambda i: (0, 0)),
            pl.BlockSpec((1, 4 * half), lambda i: (0, 0)),
        ],
        out_specs=pl.BlockSpec((1, j, _NS, 2 * _EMB), lambda i: (i, 0, 0, 0)),
        out_shape=jax.ShapeDtypeStruct((b, j, _NS, 2 * _EMB), jnp.float32),
        compiler_params=pltpu.CompilerParams(
            dimension_semantics=("parallel",)),
    )(hm, joint_table, fr128)
    return jnp.transpose(out, (0, 2, 1, 3))
